# baseline (speedup 1.0000x reference)
import functools

import jax
import jax.numpy as jnp
from jax import lax
from jax.experimental import pallas as pl
from jax.experimental.pallas import tpu as pltpu
from jax.experimental.pallas import tpu_sc as plsc

D_K = 64
NNZ = 26
NC = 2
NS = 16
NW = NC * NS
ROWS_PER_CHUNK = 4
IDX_PER_CHUNK = ROWS_PER_CHUNK * NNZ
NBUF = 4

B_U = 16384
B_V = 4096
BW_U = B_U // NW
BW_V = B_V // NW
CH_U = BW_U // ROWS_PER_CHUNK
CH_V = BW_V // ROWS_PER_CHUNK


def _make_kernel():
    mesh = plsc.VectorSubcoreMesh(core_axis_name="c", subcore_axis_name="s")

    @functools.partial(
        pl.kernel,
        out_type=(
            jax.ShapeDtypeStruct((B_U, D_K), jnp.float32),
            jax.ShapeDtypeStruct((B_V, D_K), jnp.float32),
        ),
        mesh=mesh,
        compiler_params=pltpu.CompilerParams(use_tc_tiling_on_sc=False),
        scratch_types=[
            pltpu.VMEM((BW_U, NNZ), jnp.int32),
            pltpu.VMEM((BW_U * NNZ,), jnp.int32),
            pltpu.VMEM((NBUF, IDX_PER_CHUNK, D_K), jnp.float32),
            pltpu.VMEM((BW_U, D_K), jnp.float32),
            pltpu.SemaphoreType.DMA((NBUF,)),
        ],
    )
    def fm2(u_hbm, v_hbm, wu_hbm, wv_hbm, p_hbm, q_hbm,
            idx2d_v, idx1d_v, bufs_v, out_v, sems):
        wid = lax.axis_index("s") * NC + lax.axis_index("c")

        def run_table(tbl_hbm, idx_hbm, out_hbm, n_chunks, bw):
            pltpu.sync_copy(
                idx_hbm.at[pl.ds(wid * bw, bw)], idx2d_v.at[pl.ds(0, bw)]
            )

            def repack_body(r, carry):
                lo = idx2d_v[r, pl.ds(0, 16)]
                hi = idx2d_v[r, pl.ds(NNZ - 16, 16)]
                idx1d_v[pl.ds(r * NNZ, 16)] = lo
                idx1d_v[pl.ds(r * NNZ + NNZ - 16, 16)] = hi
                return carry

            lax.fori_loop(0, bw, repack_body, 0)

            def start(g, b):
                pltpu.async_copy(
                    tbl_hbm.at[idx1d_v.at[pl.ds(g * IDX_PER_CHUNK, IDX_PER_CHUNK)]],
                    bufs_v.at[b],
                    sems.at[b],
                )

            for b in range(NBUF - 1):
                start(b, b)

            def outer_body(go, carry):
                for b in range(NBUF):
                    g = go * NBUF + b
                    s = g + NBUF - 1
                    sb = (b + NBUF - 1) % NBUF

                    @pl.when(s < n_chunks)
                    def _():
                        start(s, sb)

                    pltpu.make_async_copy(
                        tbl_hbm.at[idx1d_v.at[pl.ds(g * IDX_PER_CHUNK, IDX_PER_CHUNK)]],
                        bufs_v.at[b],
                        sems.at[b],
                    ).wait()
                    for r in range(ROWS_PER_CHUNK):
                        row = g * ROWS_PER_CHUNK + r
                        for v in range(D_K // 16):
                            acc = bufs_v[b, r * NNZ, pl.ds(v * 16, 16)]
                            for j in range(1, NNZ):
                                acc = acc + bufs_v[b, r * NNZ + j, pl.ds(v * 16, 16)]
                            out_v[row, pl.ds(v * 16, 16)] = acc
                return carry

            lax.fori_loop(0, n_chunks // NBUF, outer_body, 0)
            pltpu.sync_copy(
                out_v.at[pl.ds(0, bw)], out_hbm.at[pl.ds(wid * bw, bw)]
            )

        run_table(wu_hbm, u_hbm, p_hbm, CH_U, BW_U)
        run_table(wv_hbm, v_hbm, q_hbm, CH_V, BW_V)

    return fm2


_FM2 = _make_kernel()


@jax.jit
def kernel(U, V, W_u, W_v):
    return _FM2(U.astype(jnp.int32), V.astype(jnp.int32), W_u, W_v)


# speedup vs baseline: 1.0087x; 1.0087x over previous
import functools

import jax
import jax.numpy as jnp
from jax import lax
from jax.experimental import pallas as pl
from jax.experimental.pallas import tpu as pltpu
from jax.experimental.pallas import tpu_sc as plsc

D_U = 1000000
D_V = 100000
D_K = 64
NNZ = 26
NC = 2
NS = 16
NW = NC * NS
ROWS_PER_CHUNK = 4
IDX_PER_CHUNK = ROWS_PER_CHUNK * NNZ
NBUF = 4

B_U = 16384
B_V = 4096
BW_U = B_U // NW
BW_V = B_V // NW
CH_U = BW_U // ROWS_PER_CHUNK
CH_V = BW_V // ROWS_PER_CHUNK


def _make_kernel():
    mesh = plsc.VectorSubcoreMesh(core_axis_name="c", subcore_axis_name="s")

    @functools.partial(
        pl.kernel,
        out_type=(
            jax.ShapeDtypeStruct((B_U, D_K), jnp.float32),
            jax.ShapeDtypeStruct((B_V, D_K), jnp.float32),
        ),
        mesh=mesh,
        compiler_params=pltpu.CompilerParams(use_tc_tiling_on_sc=False),
        scratch_types=[
            pltpu.VMEM((BW_U * NNZ,), jnp.int32),
            pltpu.VMEM((NBUF, IDX_PER_CHUNK, D_K), jnp.float32),
            pltpu.VMEM((BW_U, D_K), jnp.float32),
            pltpu.SemaphoreType.DMA((NBUF,)),
        ],
    )
    def fm2(u_hbm, v_hbm, wu_hbm, wv_hbm, p_hbm, q_hbm,
            idx1d_v, bufs_v, out_v, sems):
        wid = lax.axis_index("s") * NC + lax.axis_index("c")

        def run_table(tbl_hbm, idx_hbm, out_hbm, n_chunks, bw):
            nidx = bw * NNZ
            pltpu.sync_copy(
                idx_hbm.at[pl.ds(wid * nidx, nidx)], idx1d_v.at[pl.ds(0, nidx)]
            )

            def start(g, b):
                pltpu.async_copy(
                    tbl_hbm.at[idx1d_v.at[pl.ds(g * IDX_PER_CHUNK, IDX_PER_CHUNK)]],
                    bufs_v.at[b],
                    sems.at[b],
                )

            for b in range(NBUF - 1):
                start(b, b)

            def outer_body(go, carry):
                for b in range(NBUF):
                    g = go * NBUF + b
                    s = g + NBUF - 1
                    sb = (b + NBUF - 1) % NBUF

                    @pl.when(s < n_chunks)
                    def _():
                        start(s, sb)

                    pltpu.make_async_copy(
                        tbl_hbm.at[idx1d_v.at[pl.ds(g * IDX_PER_CHUNK, IDX_PER_CHUNK)]],
                        bufs_v.at[b],
                        sems.at[b],
                    ).wait()
                    for r in range(ROWS_PER_CHUNK):
                        row = g * ROWS_PER_CHUNK + r
                        for v in range(D_K // 16):
                            acc = bufs_v[b, r * NNZ, pl.ds(v * 16, 16)]
                            for j in range(1, NNZ):
                                acc = acc + bufs_v[b, r * NNZ + j, pl.ds(v * 16, 16)]
                            out_v[row, pl.ds(v * 16, 16)] = acc
                return carry

            lax.fori_loop(0, n_chunks // NBUF, outer_body, 0)
            pltpu.sync_copy(
                out_v.at[pl.ds(0, bw)], out_hbm.at[pl.ds(wid * bw, bw)]
            )

        run_table(wu_hbm, u_hbm, p_hbm, CH_U, BW_U)
        run_table(wv_hbm, v_hbm, q_hbm, CH_V, BW_V)

    return fm2


_FM2 = _make_kernel()


@jax.jit
def kernel(U, V, W_u, W_v):
    u_flat = lax.optimization_barrier(U.astype(jnp.int32).reshape(-1))
    v_flat = lax.optimization_barrier(V.astype(jnp.int32).reshape(-1))
    wu_lin = lax.optimization_barrier(W_u.reshape(-1)).reshape(D_U, D_K)
    wv_lin = lax.optimization_barrier(W_v.reshape(-1)).reshape(D_V, D_K)
    return _FM2(u_flat, v_flat, wu_lin, wv_lin)
